# baseline (speedup 1.0000x reference)
import functools

import jax
import jax.numpy as jnp
from jax import lax
from jax.experimental import pallas as pl
from jax.experimental.pallas import tpu as pltpu
from jax.experimental.pallas import tpu_sc as plsc

_B = 16384
_D = 32
_V = 100000
_NC = 2
_NS = 16
_NW = _NC * _NS
_L = 16
_CB = 4096
_NCHUNK = _B // _CB
_UNROLL = 16


@functools.partial(
    pl.kernel,
    mesh=plsc.VectorSubcoreMesh(core_axis_name="c", subcore_axis_name="s"),
    out_type=jax.ShapeDtypeStruct((_D, _B), jnp.float32),
    scratch_types=[
        pltpu.VMEM((_V,), jnp.float32),
        pltpu.VMEM((2, _CB), jnp.int32),
        pltpu.VMEM((2, _CB), jnp.float32),
        pltpu.SemaphoreType.DMA,
        pltpu.SemaphoreType.DMA,
        pltpu.SemaphoreType.DMA,
    ],
    compiler_params=pltpu.CompilerParams(
        use_tc_tiling_on_sc=True, needs_layout_passes=False
    ),
)
def _sc_gather(idx_hbm, tab_hbm, out_hbm, col_v, idx_v, out_v, tsem, isem, osem):
    wid = lax.axis_index("c") * _NS + lax.axis_index("s")
    tab_cp = pltpu.async_copy(tab_hbm.at[wid, :], col_v, tsem)
    pltpu.async_copy(idx_hbm.at[pl.ds(0, _CB)], idx_v.at[0], isem)
    tab_cp.wait()

    def run_chunk(k, buf, nbuf):
        pltpu.make_async_copy(
            idx_hbm.at[pl.ds(0, _CB)], idx_v.at[buf], isem
        ).wait()
        if k + 1 < _NCHUNK:
            pltpu.async_copy(
                idx_hbm.at[pl.ds((k + 1) * _CB, _CB)], idx_v.at[nbuf], isem
            )
        if k >= 2:
            pltpu.make_async_copy(
                out_v.at[buf], out_hbm.at[wid, pl.ds(0, _CB)], osem
            ).wait()

        @plsc.parallel_loop(0, _CB, _L, unroll=_UNROLL)
        def _gather(o):
            iv = idx_v[buf, pl.ds(o, _L)]
            out_v[buf, pl.ds(o, _L)] = plsc.load_gather(col_v, [iv])
        pltpu.async_copy(
            out_v.at[buf], out_hbm.at[wid, pl.ds(k * _CB, _CB)], osem
        )

    for k in range(_NCHUNK):
        run_chunk(k, k % 2, (k + 1) % 2)
    for k in range(_NCHUNK - 2, _NCHUNK):
        pltpu.make_async_copy(
            out_v.at[k % 2], out_hbm.at[wid, pl.ds(0, _CB)], osem
        ).wait()


def kernel(skills, table):
    idx = skills.astype(jnp.int32)
    out_t = _sc_gather(idx, table.T)
    return out_t.T


# speedup vs baseline: 1.0166x; 1.0166x over previous
import functools

import jax
import jax.numpy as jnp
from jax import lax
from jax.experimental import pallas as pl
from jax.experimental.pallas import tpu as pltpu
from jax.experimental.pallas import tpu_sc as plsc

_B = 16384
_D = 32
_V = 100000
_NC = 2
_NS = 16
_NW = _NC * _NS
_L = 16
_CB = 4096
_NCHUNK = _B // _CB
_UNROLL = 8


@functools.partial(
    pl.kernel,
    mesh=plsc.VectorSubcoreMesh(core_axis_name="c", subcore_axis_name="s"),
    out_type=jax.ShapeDtypeStruct((_D, _B), jnp.float32),
    scratch_types=[
        pltpu.VMEM((_V,), jnp.float32),
        pltpu.VMEM((2, _CB), jnp.int32),
        pltpu.VMEM((2, _CB), jnp.float32),
        pltpu.SemaphoreType.DMA,
        pltpu.SemaphoreType.DMA,
        pltpu.SemaphoreType.DMA,
    ],
    compiler_params=pltpu.CompilerParams(
        use_tc_tiling_on_sc=True, needs_layout_passes=False,
        disable_bounds_checks=True
    ),
)
def _sc_gather(idx_hbm, tab_hbm, out_hbm, col_v, idx_v, out_v, tsem, isem, osem):
    wid = lax.axis_index("c") * _NS + lax.axis_index("s")
    tab_cp = pltpu.async_copy(tab_hbm.at[wid, :], col_v, tsem)
    pltpu.async_copy(idx_hbm.at[pl.ds(0, _CB)], idx_v.at[0], isem)
    tab_cp.wait()

    def run_chunk(k, buf, nbuf):
        pltpu.make_async_copy(
            idx_hbm.at[pl.ds(0, _CB)], idx_v.at[buf], isem
        ).wait()
        if k + 1 < _NCHUNK:
            pltpu.async_copy(
                idx_hbm.at[pl.ds((k + 1) * _CB, _CB)], idx_v.at[nbuf], isem
            )
        if k >= 2:
            pltpu.make_async_copy(
                out_v.at[buf], out_hbm.at[wid, pl.ds(0, _CB)], osem
            ).wait()

        @plsc.parallel_loop(0, _CB, _L, unroll=_UNROLL)
        def _gather(o):
            iv = idx_v[buf, pl.ds(o, _L)]
            out_v[buf, pl.ds(o, _L)] = plsc.load_gather(col_v, [iv])
        pltpu.async_copy(
            out_v.at[buf], out_hbm.at[wid, pl.ds(k * _CB, _CB)], osem
        )

    for k in range(_NCHUNK):
        run_chunk(k, k % 2, (k + 1) % 2)
    for k in range(_NCHUNK - 2, _NCHUNK):
        pltpu.make_async_copy(
            out_v.at[k % 2], out_hbm.at[wid, pl.ds(0, _CB)], osem
        ).wait()


def kernel(skills, table):
    idx = skills.astype(jnp.int32)
    out_t = _sc_gather(idx, table.T)
    return out_t.T
